# baseline (speedup 1.0000x reference)
import jax
import jax.numpy as jnp
from jax import lax
from jax.experimental import pallas as pl
from jax.experimental.pallas import tpu as pltpu
from jax.experimental.pallas import tpu_sc as plsc

N_NODES = 10000
HIDDEN = 128
NUM_CLASSES = 4
N_EDGES = 320000

NUM_CORES = 2
NUM_SUBCORES = 16
NUM_TILES = NUM_CORES * NUM_SUBCORES
EDGES_PER_TILE = N_EDGES // NUM_TILES
CHUNK = 80
CHUNKS_PER_TILE = EDGES_PER_TILE // CHUNK
ACC_ROWS = N_NODES
ZERO_SUBCORES = 10
ZERO_ROWS = ACC_ROWS // ZERO_SUBCORES

ROW_BLOCK = 400
GRID = N_NODES // ROW_BLOCK
PART_BLOCKS = ACC_ROWS // ROW_BLOCK


def _sc_aggregate(q, src, dst, zeros, d, gather):
  mesh = plsc.VectorSubcoreMesh(core_axis_name="c", subcore_axis_name="s")

  def body(*refs):
    if gather:
      (q_hbm, srcr_hbm, dstr_hbm, z_hbm, out_hbm,
       srcs_v, dsts_v, rows0, rows1, acc, sem0, sem1) = refs
    else:
      dstr_hbm, z_hbm, out_hbm, dsts_v, rows0, acc, sem0 = refs
    c = lax.axis_index("c")
    s = lax.axis_index("s")
    wid = c * NUM_SUBCORES + s

    r0 = s * ZERO_ROWS

    @pl.when(s < ZERO_SUBCORES)
    def _zero():
      pltpu.sync_copy(z_hbm.at[pl.ds(r0, ZERO_ROWS)],
                      acc.at[pl.ds(r0, ZERO_ROWS)])

    t0 = wid * CHUNKS_PER_TILE
    pltpu.sync_copy(dstr_hbm.at[pl.ds(t0, CHUNKS_PER_TILE)], dsts_v)
    if gather:
      pltpu.sync_copy(srcr_hbm.at[pl.ds(t0, CHUNKS_PER_TILE)], srcs_v)
    else:
      @pl.loop(0, CHUNK)
      def _fill(r):
        for j in range(d // 16):
          rows0[r, pl.ds(j * 16, 16)] = jnp.ones((16,), jnp.float32)

    plsc.subcore_barrier()

    if gather:
      pltpu.async_copy(q_hbm.at[srcs_v.at[0]], rows0, sem0)

      @pl.loop(0, CHUNKS_PER_TILE - 1, step=2)
      def _edge_pair(j):
        pltpu.async_copy(q_hbm.at[srcs_v.at[j + 1]], rows1, sem1)
        pltpu.make_async_copy(q_hbm.at[srcs_v.at[j]], rows0, sem0).wait()
        pltpu.sync_copy(rows0, acc.at[dsts_v.at[j]], add=True)
        pltpu.async_copy(q_hbm.at[srcs_v.at[j + 2]], rows0, sem0)
        pltpu.make_async_copy(q_hbm.at[srcs_v.at[j + 1]], rows1, sem1).wait()
        pltpu.sync_copy(rows1, acc.at[dsts_v.at[j + 1]], add=True)

      last = CHUNKS_PER_TILE - 1
      pltpu.make_async_copy(q_hbm.at[srcs_v.at[last]], rows0, sem0).wait()
      pltpu.sync_copy(rows0, acc.at[dsts_v.at[last]], add=True)
    else:
      @pl.loop(0, CHUNKS_PER_TILE, step=5)
      def _deg_batch(j):
        for b in range(5):
          pltpu.async_copy(rows0, acc.at[dsts_v.at[j + b]], sem0, add=True)
        for b in range(5):
          pltpu.make_async_copy(rows0, acc.at[dsts_v.at[j + b]], sem0).wait()

    plsc.subcore_barrier()

    @pl.when(s < ZERO_SUBCORES)
    def _writeback():
      pltpu.sync_copy(acc.at[pl.ds(r0, ZERO_ROWS)],
                      out_hbm.at[pl.ds(c * ACC_ROWS + r0, ZERO_ROWS)])

  if gather:
    scratch = [
        pltpu.VMEM((CHUNKS_PER_TILE, CHUNK), jnp.int32),
        pltpu.VMEM((CHUNKS_PER_TILE, CHUNK), jnp.int32),
        pltpu.VMEM((CHUNK, d), jnp.float32),
        pltpu.VMEM((CHUNK, d), jnp.float32),
        pltpu.VMEM_SHARED((ACC_ROWS, d), jnp.float32),
        pltpu.SemaphoreType.DMA,
        pltpu.SemaphoreType.DMA,
    ]
  else:
    scratch = [
        pltpu.VMEM((CHUNKS_PER_TILE, CHUNK), jnp.int32),
        pltpu.VMEM((CHUNK, d), jnp.float32),
        pltpu.VMEM_SHARED((ACC_ROWS, d), jnp.float32),
        pltpu.SemaphoreType.DMA,
    ]
  k = pl.kernel(
      body,
      out_type=jax.ShapeDtypeStruct((NUM_CORES * ACC_ROWS, d), jnp.float32),
      mesh=mesh,
      scratch_types=scratch,
      compiler_params=pltpu.CompilerParams(use_tc_tiling_on_sc=False),
  )
  if gather:
    return k(q, src, dst, zeros)
  return k(dst, zeros)


def _tc_embed(x3, degp, W_pre, b_pre, W1):
  def body(deg0_ref, deg1_ref, x_ref, wp_ref, bp_ref, w1_ref, q1_ref, dinv_ref):
    deg = deg0_ref[:, 0] + deg1_ref[:, 0] + 1.0
    dinv = lax.rsqrt(deg)
    xr = x_ref[...].reshape(ROW_BLOCK, 4096)
    h = jnp.maximum(
        jnp.dot(xr, wp_ref[...], preferred_element_type=jnp.float32)
        + bp_ref[...], 0.0)
    p = jnp.dot(h, w1_ref[...], preferred_element_type=jnp.float32)
    q1_ref[...] = p * dinv[:, None]
    dinv_ref[...] = dinv[:, None]

  return pl.pallas_call(
      body,
      grid=(GRID,),
      in_specs=[
          pl.BlockSpec((ROW_BLOCK, 16), lambda i: (i, 0)),
          pl.BlockSpec((ROW_BLOCK, 16), lambda i: (i + PART_BLOCKS, 0)),
          pl.BlockSpec((ROW_BLOCK, 128, 32), lambda i: (i, 0, 0)),
          pl.BlockSpec((4096, HIDDEN), lambda i: (0, 0)),
          pl.BlockSpec((1, HIDDEN), lambda i: (0, 0)),
          pl.BlockSpec((HIDDEN, HIDDEN), lambda i: (0, 0)),
      ],
      out_specs=[
          pl.BlockSpec((ROW_BLOCK, HIDDEN), lambda i: (i, 0)),
          pl.BlockSpec((ROW_BLOCK, 1), lambda i: (i, 0)),
      ],
      out_shape=[
          jax.ShapeDtypeStruct((N_NODES, HIDDEN), jnp.float32),
          jax.ShapeDtypeStruct((N_NODES, 1), jnp.float32),
      ],
      compiler_params=pltpu.CompilerParams(
          dimension_semantics=("arbitrary",)),
  )(degp, degp, x3, W_pre, b_pre, W1)


def _tc_mid(aggp1, q1, dinv, W2p, b1):
  def body(a0_ref, a1_ref, q1_ref, dinv_ref, w2_ref, b1_ref, q2_ref):
    s = a0_ref[...] + a1_ref[...] + q1_ref[...]
    dinv = dinv_ref[...]
    h1 = jnp.maximum(s * dinv + b1_ref[...], 0.0)
    q2_ref[...] = jnp.dot(
        h1, w2_ref[...], preferred_element_type=jnp.float32) * dinv

  return pl.pallas_call(
      body,
      grid=(GRID,),
      in_specs=[
          pl.BlockSpec((ROW_BLOCK, HIDDEN), lambda i: (i, 0)),
          pl.BlockSpec((ROW_BLOCK, HIDDEN), lambda i: (i + PART_BLOCKS, 0)),
          pl.BlockSpec((ROW_BLOCK, HIDDEN), lambda i: (i, 0)),
          pl.BlockSpec((ROW_BLOCK, 1), lambda i: (i, 0)),
          pl.BlockSpec((HIDDEN, 16), lambda i: (0, 0)),
          pl.BlockSpec((1, HIDDEN), lambda i: (0, 0)),
      ],
      out_specs=pl.BlockSpec((ROW_BLOCK, 16), lambda i: (i, 0)),
      out_shape=jax.ShapeDtypeStruct((N_NODES, 16), jnp.float32),
      compiler_params=pltpu.CompilerParams(
          dimension_semantics=("arbitrary",)),
  )(aggp1, aggp1, q1, dinv, W2p, b1)


def _tc_final(aggp2, q2, dinv, b2):
  def body(a0_ref, a1_ref, q2_ref, dinv_ref, b2_ref, out_ref):
    s = a0_ref[...] + a1_ref[...] + q2_ref[...]
    out_ref[...] = s[:, :NUM_CLASSES] * dinv_ref[...] + b2_ref[...]

  return pl.pallas_call(
      body,
      grid=(GRID,),
      in_specs=[
          pl.BlockSpec((ROW_BLOCK, 16), lambda i: (i, 0)),
          pl.BlockSpec((ROW_BLOCK, 16), lambda i: (i + PART_BLOCKS, 0)),
          pl.BlockSpec((ROW_BLOCK, 16), lambda i: (i, 0)),
          pl.BlockSpec((ROW_BLOCK, 1), lambda i: (i, 0)),
          pl.BlockSpec((1, NUM_CLASSES), lambda i: (0, 0)),
      ],
      out_specs=pl.BlockSpec((ROW_BLOCK, NUM_CLASSES), lambda i: (i, 0)),
      out_shape=jax.ShapeDtypeStruct((N_NODES, NUM_CLASSES), jnp.float32),
      compiler_params=pltpu.CompilerParams(
          dimension_semantics=("arbitrary",)),
  )(aggp2, aggp2, q2, dinv, b2)


def kernel(x, edge_index, W_pre, b_pre, W1, b1, W2, b2):
  src = edge_index[0].astype(jnp.int32).reshape(N_EDGES // CHUNK, CHUNK)
  dst = edge_index[1].astype(jnp.int32).reshape(N_EDGES // CHUNK, CHUNK)

  zeros16 = jnp.zeros((ACC_ROWS, 16), jnp.float32)
  zeros128 = jnp.zeros((ACC_ROWS, HIDDEN), jnp.float32)
  W2p = jnp.pad(W2, ((0, 0), (0, 16 - NUM_CLASSES)))
  b_pre2 = b_pre.reshape(1, HIDDEN)
  b1_2 = b1.reshape(1, HIDDEN)
  b2_2 = b2.reshape(1, NUM_CLASSES)

  degp = _sc_aggregate(None, None, dst, zeros16, 16, gather=False)
  q1, dinv = _tc_embed(x, degp, W_pre, b_pre2, W1)
  aggp1 = _sc_aggregate(q1, src, dst, zeros128, HIDDEN, gather=True)
  q2 = _tc_mid(aggp1, q1, dinv, W2p, b1_2)
  aggp2 = _sc_aggregate(q2, src, dst, zeros16, 16, gather=True)
  return _tc_final(aggp2, q2, dinv, b2_2)


# speedup vs baseline: 1.7010x; 1.7010x over previous
import jax
import jax.numpy as jnp
from jax import lax
from jax.experimental import pallas as pl
from jax.experimental.pallas import tpu as pltpu
from jax.experimental.pallas import tpu_sc as plsc

N_NODES = 10000
HIDDEN = 128
NUM_CLASSES = 4
N_EDGES = 320000

NUM_CORES = 2
NUM_SUBCORES = 16
NUM_TILES = NUM_CORES * NUM_SUBCORES
EDGES_PER_TILE = N_EDGES // NUM_TILES
CHUNK = 80
CHUNKS_PER_TILE = EDGES_PER_TILE // CHUNK
ACC_ROWS = N_NODES
ZERO_SUBCORES = 10
ZERO_ROWS = ACC_ROWS // ZERO_SUBCORES

ROW_BLOCK = 400
GRID = N_NODES // ROW_BLOCK
PART_BLOCKS = ACC_ROWS // ROW_BLOCK


def _sc_aggregate(q, src, dst, zeros, d, gather):
  mesh = plsc.VectorSubcoreMesh(core_axis_name="c", subcore_axis_name="s")

  def body(*refs):
    if gather:
      (q_hbm, srcr_hbm, dstr_hbm, z_hbm, out_hbm,
       srcs_v, dsts_v, rows0, rows1, acc, sem0, sem1) = refs
    else:
      dstr_hbm, z_hbm, out_hbm, dsts_v, rows0, acc, sem0 = refs
    c = lax.axis_index("c")
    s = lax.axis_index("s")
    wid = c * NUM_SUBCORES + s

    r0 = s * ZERO_ROWS

    @pl.when(s < ZERO_SUBCORES)
    def _zero():
      pltpu.sync_copy(z_hbm.at[pl.ds(r0, ZERO_ROWS)],
                      acc.at[pl.ds(r0, ZERO_ROWS)])

    t0 = wid * CHUNKS_PER_TILE
    pltpu.sync_copy(dstr_hbm.at[pl.ds(t0, CHUNKS_PER_TILE)], dsts_v)
    if gather:
      pltpu.sync_copy(srcr_hbm.at[pl.ds(t0, CHUNKS_PER_TILE)], srcs_v)
    else:
      @pl.loop(0, CHUNK)
      def _fill(r):
        for j in range(d // 16):
          rows0[r, pl.ds(j * 16, 16)] = jnp.ones((16,), jnp.float32)

    plsc.subcore_barrier()

    if gather:
      pltpu.async_copy(q_hbm.at[srcs_v.at[0]], rows0, sem0)

      @pl.loop(0, CHUNKS_PER_TILE - 1, step=2)
      def _edge_pair(j):
        pltpu.async_copy(q_hbm.at[srcs_v.at[j + 1]], rows1, sem1)
        pltpu.make_async_copy(q_hbm.at[srcs_v.at[j]], rows0, sem0).wait()
        pltpu.sync_copy(rows0, acc.at[dsts_v.at[j]], add=True)
        pltpu.async_copy(q_hbm.at[srcs_v.at[j + 2]], rows0, sem0)
        pltpu.make_async_copy(q_hbm.at[srcs_v.at[j + 1]], rows1, sem1).wait()
        pltpu.sync_copy(rows1, acc.at[dsts_v.at[j + 1]], add=True)

      last = CHUNKS_PER_TILE - 1
      pltpu.make_async_copy(q_hbm.at[srcs_v.at[last]], rows0, sem0).wait()
      pltpu.sync_copy(rows0, acc.at[dsts_v.at[last]], add=True)
    else:
      @pl.loop(0, CHUNKS_PER_TILE, step=5)
      def _deg_batch(j):
        for b in range(5):
          pltpu.async_copy(rows0, acc.at[dsts_v.at[j + b]], sem0, add=True)
        for b in range(5):
          pltpu.make_async_copy(rows0, acc.at[dsts_v.at[j + b]], sem0).wait()

    plsc.subcore_barrier()

    @pl.when(s < ZERO_SUBCORES)
    def _writeback():
      pltpu.sync_copy(acc.at[pl.ds(r0, ZERO_ROWS)],
                      out_hbm.at[pl.ds(c * ACC_ROWS + r0, ZERO_ROWS)])

  if gather:
    scratch = [
        pltpu.VMEM((CHUNKS_PER_TILE, CHUNK), jnp.int32),
        pltpu.VMEM((CHUNKS_PER_TILE, CHUNK), jnp.int32),
        pltpu.VMEM((CHUNK, d), jnp.float32),
        pltpu.VMEM((CHUNK, d), jnp.float32),
        pltpu.VMEM_SHARED((ACC_ROWS, d), jnp.float32),
        pltpu.SemaphoreType.DMA,
        pltpu.SemaphoreType.DMA,
    ]
  else:
    scratch = [
        pltpu.VMEM((CHUNKS_PER_TILE, CHUNK), jnp.int32),
        pltpu.VMEM((CHUNK, d), jnp.float32),
        pltpu.VMEM_SHARED((ACC_ROWS, d), jnp.float32),
        pltpu.SemaphoreType.DMA,
    ]
  k = pl.kernel(
      body,
      out_type=jax.ShapeDtypeStruct((NUM_CORES * ACC_ROWS, d), jnp.float32),
      mesh=mesh,
      scratch_types=scratch,
      compiler_params=pltpu.CompilerParams(use_tc_tiling_on_sc=False),
  )
  if gather:
    return k(q, src, dst, zeros)
  return k(dst, zeros)


def _tc_embed(xf, degp, W_pre, b_pre, W1):
  def body(deg0_ref, deg1_ref, x_ref, wp_ref, bp_ref, w1_ref, q1_ref, dinv_ref):
    deg = deg0_ref[:, 0] + deg1_ref[:, 0] + 1.0
    dinv = lax.rsqrt(deg)
    h = jnp.maximum(
        jnp.dot(x_ref[...], wp_ref[...], preferred_element_type=jnp.float32)
        + bp_ref[...], 0.0)
    p = jnp.dot(h, w1_ref[...], preferred_element_type=jnp.float32)
    q1_ref[...] = p * dinv[:, None]
    dinv_ref[...] = dinv[:, None]

  return pl.pallas_call(
      body,
      grid=(GRID,),
      in_specs=[
          pl.BlockSpec((ROW_BLOCK, 16), lambda i: (i, 0)),
          pl.BlockSpec((ROW_BLOCK, 16), lambda i: (i + PART_BLOCKS, 0)),
          pl.BlockSpec((ROW_BLOCK, 4096), lambda i: (i, 0)),
          pl.BlockSpec((4096, HIDDEN), lambda i: (0, 0)),
          pl.BlockSpec((1, HIDDEN), lambda i: (0, 0)),
          pl.BlockSpec((HIDDEN, HIDDEN), lambda i: (0, 0)),
      ],
      out_specs=[
          pl.BlockSpec((ROW_BLOCK, HIDDEN), lambda i: (i, 0)),
          pl.BlockSpec((ROW_BLOCK, 1), lambda i: (i, 0)),
      ],
      out_shape=[
          jax.ShapeDtypeStruct((N_NODES, HIDDEN), jnp.float32),
          jax.ShapeDtypeStruct((N_NODES, 1), jnp.float32),
      ],
      compiler_params=pltpu.CompilerParams(
          dimension_semantics=("arbitrary",)),
  )(degp, degp, xf, W_pre, b_pre, W1)


def _tc_mid(aggp1, q1, dinv, W2p, b1):
  def body(a0_ref, a1_ref, q1_ref, dinv_ref, w2_ref, b1_ref, q2_ref):
    s = a0_ref[...] + a1_ref[...] + q1_ref[...]
    dinv = dinv_ref[...]
    h1 = jnp.maximum(s * dinv + b1_ref[...], 0.0)
    q2_ref[...] = jnp.dot(
        h1, w2_ref[...], preferred_element_type=jnp.float32) * dinv

  return pl.pallas_call(
      body,
      grid=(GRID,),
      in_specs=[
          pl.BlockSpec((ROW_BLOCK, HIDDEN), lambda i: (i, 0)),
          pl.BlockSpec((ROW_BLOCK, HIDDEN), lambda i: (i + PART_BLOCKS, 0)),
          pl.BlockSpec((ROW_BLOCK, HIDDEN), lambda i: (i, 0)),
          pl.BlockSpec((ROW_BLOCK, 1), lambda i: (i, 0)),
          pl.BlockSpec((HIDDEN, 16), lambda i: (0, 0)),
          pl.BlockSpec((1, HIDDEN), lambda i: (0, 0)),
      ],
      out_specs=pl.BlockSpec((ROW_BLOCK, 16), lambda i: (i, 0)),
      out_shape=jax.ShapeDtypeStruct((N_NODES, 16), jnp.float32),
      compiler_params=pltpu.CompilerParams(
          dimension_semantics=("arbitrary",)),
  )(aggp1, aggp1, q1, dinv, W2p, b1)


def _tc_final(aggp2, q2, dinv, b2):
  def body(a0_ref, a1_ref, q2_ref, dinv_ref, b2_ref, out_ref):
    s = a0_ref[...] + a1_ref[...] + q2_ref[...]
    out_ref[...] = s[:, :NUM_CLASSES] * dinv_ref[...] + b2_ref[...]

  return pl.pallas_call(
      body,
      grid=(GRID,),
      in_specs=[
          pl.BlockSpec((ROW_BLOCK, 16), lambda i: (i, 0)),
          pl.BlockSpec((ROW_BLOCK, 16), lambda i: (i + PART_BLOCKS, 0)),
          pl.BlockSpec((ROW_BLOCK, 16), lambda i: (i, 0)),
          pl.BlockSpec((ROW_BLOCK, 1), lambda i: (i, 0)),
          pl.BlockSpec((1, NUM_CLASSES), lambda i: (0, 0)),
      ],
      out_specs=pl.BlockSpec((ROW_BLOCK, NUM_CLASSES), lambda i: (i, 0)),
      out_shape=jax.ShapeDtypeStruct((N_NODES, NUM_CLASSES), jnp.float32),
      compiler_params=pltpu.CompilerParams(
          dimension_semantics=("arbitrary",)),
  )(aggp2, aggp2, q2, dinv, b2)


def kernel(x, edge_index, W_pre, b_pre, W1, b1, W2, b2):
  n = x.shape[0]
  xf = x.reshape(n, -1).astype(jnp.bfloat16)
  src = edge_index[0].astype(jnp.int32).reshape(N_EDGES // CHUNK, CHUNK)
  dst = edge_index[1].astype(jnp.int32).reshape(N_EDGES // CHUNK, CHUNK)

  zeros16 = jnp.zeros((ACC_ROWS, 16), jnp.float32)
  zeros128 = jnp.zeros((ACC_ROWS, HIDDEN), jnp.float32)
  W2p = jnp.pad(W2, ((0, 0), (0, 16 - NUM_CLASSES)))
  b_pre2 = b_pre.reshape(1, HIDDEN)
  b1_2 = b1.reshape(1, HIDDEN)
  b2_2 = b2.reshape(1, NUM_CLASSES)

  degp = _sc_aggregate(None, None, dst, zeros16, 16, gather=False)
  q1, dinv = _tc_embed(xf, degp, W_pre.astype(jnp.bfloat16), b_pre2, W1)
  aggp1 = _sc_aggregate(q1, src, dst, zeros128, HIDDEN, gather=True)
  q2 = _tc_mid(aggp1, q1, dinv, W2p, b1_2)
  aggp2 = _sc_aggregate(q2, src, dst, zeros16, 16, gather=True)
  return _tc_final(aggp2, q2, dinv, b2_2)


# speedup vs baseline: 1.7209x; 1.0117x over previous
import jax
import jax.numpy as jnp
from jax import lax
from jax.experimental import pallas as pl
from jax.experimental.pallas import tpu as pltpu
from jax.experimental.pallas import tpu_sc as plsc

N_NODES = 10000
HIDDEN = 128
NUM_CLASSES = 4
N_EDGES = 320000

NUM_CORES = 2
NUM_SUBCORES = 16
NUM_TILES = NUM_CORES * NUM_SUBCORES
EDGES_PER_TILE = N_EDGES // NUM_TILES
CHUNK = 80
CHUNKS_PER_TILE = EDGES_PER_TILE // CHUNK
IDX_GROUPS = 5
GCHUNKS = CHUNKS_PER_TILE // IDX_GROUPS
NBUF = 3
ACC_ROWS = N_NODES
ZERO_SUBCORES = 10
ZERO_ROWS = ACC_ROWS // ZERO_SUBCORES

ROW_BLOCK = 400
GRID = N_NODES // ROW_BLOCK
PART_BLOCKS = ACC_ROWS // ROW_BLOCK


def _sc_aggregate(q, src, dst, zeros, d, gather):
  mesh = plsc.VectorSubcoreMesh(core_axis_name="c", subcore_axis_name="s")

  def body(*refs):
    if gather:
      (q_hbm, srcr_hbm, dstr_hbm, z_hbm, out_hbm,
       srcs_v, dsts_v, r0b, r1b, r2b, acc,
       sg0, sg1, sg2, ss0, ss1, ss2) = refs
      rows = (r0b, r1b, r2b)
      semg = (sg0, sg1, sg2)
      sems = (ss0, ss1, ss2)
    else:
      dstr_hbm, z_hbm, out_hbm, dsts_v, r0b, acc, ss0 = refs
      rows = (r0b,)
      sems = (ss0,)
    c = lax.axis_index("c")
    s = lax.axis_index("s")
    wid = c * NUM_SUBCORES + s

    r0 = s * ZERO_ROWS

    @pl.when(s < ZERO_SUBCORES)
    def _zero():
      pltpu.sync_copy(z_hbm.at[pl.ds(r0, ZERO_ROWS)],
                      acc.at[pl.ds(r0, ZERO_ROWS)])

    if not gather:
      @pl.loop(0, CHUNK)
      def _fill(r):
        for j in range(d // 16):
          rows[0][r, pl.ds(j * 16, 16)] = jnp.ones((16,), jnp.float32)

    plsc.subcore_barrier()

    t0 = wid * CHUNKS_PER_TILE

    if gather:
      @pl.loop(0, IDX_GROUPS)
      def _group(g):
        pltpu.sync_copy(dstr_hbm.at[pl.ds(t0 + g * GCHUNKS, GCHUNKS)], dsts_v)
        pltpu.sync_copy(srcr_hbm.at[pl.ds(t0 + g * GCHUNKS, GCHUNKS)], srcs_v)
        pltpu.async_copy(q_hbm.at[srcs_v.at[0]], rows[0], semg[0])
        pltpu.async_copy(q_hbm.at[srcs_v.at[1]], rows[1], semg[1])

        @pl.loop(0, GCHUNKS // NBUF)
        def _ring(tt):
          for b in range(NBUF):
            j = tt * NBUF + b
            p = (b + 2) % NBUF
            pltpu.make_async_copy(q_hbm.at[srcs_v.at[j]], rows[b],
                                  semg[b]).wait()
            pltpu.async_copy(rows[b], acc.at[dsts_v.at[j]], sems[b], add=True)

            @pl.when(jnp.logical_and(j + 2 <= GCHUNKS - 1, j >= 1))
            def _reuse():
              pltpu.make_async_copy(rows[p], acc.at[dsts_v.at[j - 1]],
                                    sems[p]).wait()
              pltpu.async_copy(q_hbm.at[srcs_v.at[j + 2]], rows[p], semg[p])

            @pl.when(jnp.logical_and(j + 2 <= GCHUNKS - 1, j < 1))
            def _first():
              pltpu.async_copy(q_hbm.at[srcs_v.at[j + 2]], rows[p], semg[p])

        last = GCHUNKS - 1
        pltpu.make_async_copy(q_hbm.at[srcs_v.at[last]], rows[0],
                              semg[0]).wait()
        pltpu.async_copy(rows[0], acc.at[dsts_v.at[last]], sems[0], add=True)
        for b in (1, 2, 0):
          j = GCHUNKS - 3 + ((b - 1) % NBUF)
          pltpu.make_async_copy(rows[b], acc.at[dsts_v.at[j]], sems[b]).wait()
    else:
      @pl.loop(0, IDX_GROUPS)
      def _dgroup(g):
        pltpu.sync_copy(dstr_hbm.at[pl.ds(t0 + g * GCHUNKS, GCHUNKS)], dsts_v)

        @pl.loop(0, GCHUNKS, step=5)
        def _deg_batch(j):
          for b in range(5):
            pltpu.async_copy(rows[0], acc.at[dsts_v.at[j + b]], sems[0],
                             add=True)
          for b in range(5):
            pltpu.make_async_copy(rows[0], acc.at[dsts_v.at[j + b]],
                                  sems[0]).wait()

    plsc.subcore_barrier()

    @pl.when(s < ZERO_SUBCORES)
    def _writeback():
      pltpu.sync_copy(acc.at[pl.ds(r0, ZERO_ROWS)],
                      out_hbm.at[pl.ds(c * ACC_ROWS + r0, ZERO_ROWS)])

  if gather:
    scratch = [
        pltpu.VMEM((GCHUNKS, CHUNK), jnp.int32),
        pltpu.VMEM((GCHUNKS, CHUNK), jnp.int32),
        pltpu.VMEM((CHUNK, d), jnp.float32),
        pltpu.VMEM((CHUNK, d), jnp.float32),
        pltpu.VMEM((CHUNK, d), jnp.float32),
        pltpu.VMEM_SHARED((ACC_ROWS, d), jnp.float32),
    ] + [pltpu.SemaphoreType.DMA] * 6
  else:
    scratch = [
        pltpu.VMEM((GCHUNKS, CHUNK), jnp.int32),
        pltpu.VMEM((CHUNK, d), jnp.float32),
        pltpu.VMEM_SHARED((ACC_ROWS, d), jnp.float32),
        pltpu.SemaphoreType.DMA,
    ]
  k = pl.kernel(
      body,
      out_type=jax.ShapeDtypeStruct((NUM_CORES * ACC_ROWS, d), jnp.float32),
      mesh=mesh,
      scratch_types=scratch,
      compiler_params=pltpu.CompilerParams(use_tc_tiling_on_sc=False),
  )
  if gather:
    return k(q, src, dst, zeros)
  return k(dst, zeros)


def _tc_embed(xf, degp, W_pre, b_pre, W1):
  def body(deg0_ref, deg1_ref, x_ref, wp_ref, bp_ref, w1_ref, q1_ref, dinv_ref):
    deg = deg0_ref[:, 0] + deg1_ref[:, 0] + 1.0
    dinv = lax.rsqrt(deg)
    h = jnp.maximum(
        jnp.dot(x_ref[...], wp_ref[...], preferred_element_type=jnp.float32)
        + bp_ref[...], 0.0)
    p = jnp.dot(h, w1_ref[...], preferred_element_type=jnp.float32)
    q1_ref[...] = p * dinv[:, None]
    dinv_ref[...] = dinv[:, None]

  return pl.pallas_call(
      body,
      grid=(GRID,),
      in_specs=[
          pl.BlockSpec((ROW_BLOCK, 16), lambda i: (i, 0)),
          pl.BlockSpec((ROW_BLOCK, 16), lambda i: (i + PART_BLOCKS, 0)),
          pl.BlockSpec((ROW_BLOCK, 4096), lambda i: (i, 0)),
          pl.BlockSpec((4096, HIDDEN), lambda i: (0, 0)),
          pl.BlockSpec((1, HIDDEN), lambda i: (0, 0)),
          pl.BlockSpec((HIDDEN, HIDDEN), lambda i: (0, 0)),
      ],
      out_specs=[
          pl.BlockSpec((ROW_BLOCK, HIDDEN), lambda i: (i, 0)),
          pl.BlockSpec((ROW_BLOCK, 1), lambda i: (i, 0)),
      ],
      out_shape=[
          jax.ShapeDtypeStruct((N_NODES, HIDDEN), jnp.float32),
          jax.ShapeDtypeStruct((N_NODES, 1), jnp.float32),
      ],
      compiler_params=pltpu.CompilerParams(
          dimension_semantics=("arbitrary",)),
  )(degp, degp, xf, W_pre, b_pre, W1)


def _tc_mid(aggp1, q1, dinv, W2p, b1):
  def body(a0_ref, a1_ref, q1_ref, dinv_ref, w2_ref, b1_ref, q2_ref):
    s = a0_ref[...] + a1_ref[...] + q1_ref[...]
    dinv = dinv_ref[...]
    h1 = jnp.maximum(s * dinv + b1_ref[...], 0.0)
    q2_ref[...] = jnp.dot(
        h1, w2_ref[...], preferred_element_type=jnp.float32) * dinv

  return pl.pallas_call(
      body,
      grid=(GRID,),
      in_specs=[
          pl.BlockSpec((ROW_BLOCK, HIDDEN), lambda i: (i, 0)),
          pl.BlockSpec((ROW_BLOCK, HIDDEN), lambda i: (i + PART_BLOCKS, 0)),
          pl.BlockSpec((ROW_BLOCK, HIDDEN), lambda i: (i, 0)),
          pl.BlockSpec((ROW_BLOCK, 1), lambda i: (i, 0)),
          pl.BlockSpec((HIDDEN, 16), lambda i: (0, 0)),
          pl.BlockSpec((1, HIDDEN), lambda i: (0, 0)),
      ],
      out_specs=pl.BlockSpec((ROW_BLOCK, 16), lambda i: (i, 0)),
      out_shape=jax.ShapeDtypeStruct((N_NODES, 16), jnp.float32),
      compiler_params=pltpu.CompilerParams(
          dimension_semantics=("arbitrary",)),
  )(aggp1, aggp1, q1, dinv, W2p, b1)


def _tc_final(aggp2, q2, dinv, b2):
  def body(a0_ref, a1_ref, q2_ref, dinv_ref, b2_ref, out_ref):
    s = a0_ref[...] + a1_ref[...] + q2_ref[...]
    out_ref[...] = s[:, :NUM_CLASSES] * dinv_ref[...] + b2_ref[...]

  return pl.pallas_call(
      body,
      grid=(GRID,),
      in_specs=[
          pl.BlockSpec((ROW_BLOCK, 16), lambda i: (i, 0)),
          pl.BlockSpec((ROW_BLOCK, 16), lambda i: (i + PART_BLOCKS, 0)),
          pl.BlockSpec((ROW_BLOCK, 16), lambda i: (i, 0)),
          pl.BlockSpec((ROW_BLOCK, 1), lambda i: (i, 0)),
          pl.BlockSpec((1, NUM_CLASSES), lambda i: (0, 0)),
      ],
      out_specs=pl.BlockSpec((ROW_BLOCK, NUM_CLASSES), lambda i: (i, 0)),
      out_shape=jax.ShapeDtypeStruct((N_NODES, NUM_CLASSES), jnp.float32),
      compiler_params=pltpu.CompilerParams(
          dimension_semantics=("arbitrary",)),
  )(aggp2, aggp2, q2, dinv, b2)


def kernel(x, edge_index, W_pre, b_pre, W1, b1, W2, b2):
  n = x.shape[0]
  xf = x.astype(jnp.bfloat16).reshape(n, -1)
  src = edge_index[0].astype(jnp.int32).reshape(N_EDGES // CHUNK, CHUNK)
  dst = edge_index[1].astype(jnp.int32).reshape(N_EDGES // CHUNK, CHUNK)

  zeros16 = jnp.zeros((ACC_ROWS, 16), jnp.float32)
  zeros128 = jnp.zeros((ACC_ROWS, HIDDEN), jnp.float32)
  W2p = jnp.pad(W2, ((0, 0), (0, 16 - NUM_CLASSES)))
  b_pre2 = b_pre.reshape(1, HIDDEN)
  b1_2 = b1.reshape(1, HIDDEN)
  b2_2 = b2.reshape(1, NUM_CLASSES)

  degp = _sc_aggregate(None, None, dst, zeros16, 16, gather=False)
  q1, dinv = _tc_embed(xf, degp, W_pre.astype(jnp.bfloat16), b_pre2, W1)
  aggp1 = _sc_aggregate(q1, src, dst, zeros128, HIDDEN, gather=True)
  q2 = _tc_mid(aggp1, q1, dinv, W2p, b1_2)
  aggp2 = _sc_aggregate(q2, src, dst, zeros16, 16, gather=True)
  return _tc_final(aggp2, q2, dinv, b2_2)
